# baseline (speedup 1.0000x reference)
import functools

import jax
import jax.numpy as jnp
from jax import lax
from jax.experimental import pallas as pl
from jax.experimental.pallas import tpu as pltpu
from jax.experimental.pallas import tpu_sc as plsc

B = 16384
K = 64
NC = 2
NS = 16
NW = NC * NS
RPW = B // NW
CH = 256
NCHUNK = RPW // CH

_mesh = plsc.VectorSubcoreMesh(core_axis_name="c", subcore_axis_name="s")


@functools.partial(
    pl.kernel,
    mesh=_mesh,
    out_type=jax.ShapeDtypeStruct((B,), jnp.float32),
    scratch_types=[
        pltpu.VMEM((CH * K,), jnp.float32),
        pltpu.VMEM((CH * K,), jnp.float32),
        pltpu.VMEM((CH * K,), jnp.float32),
        pltpu.VMEM((CH * K,), jnp.float32),
        pltpu.VMEM((RPW,), jnp.float32),
        pltpu.VMEM((RPW,), jnp.float32),
        pltpu.VMEM((RPW,), jnp.float32),
        pltpu.VMEM((RPW,), jnp.float32),
        pltpu.VMEM((RPW,), jnp.float32),
        pltpu.VMEM((16,), jnp.float32),
        pltpu.VMEM((256,), jnp.float32),
    ],
)
def _sc_kernel(gu_h, gi_h, gut_h, git_h, bu_h, bi_h, but_h, bit_h, mu_h,
               out_h, gu_v, gi_v, gut_v, git_v,
               bu_v, bi_v, but_v, bit_v, out_v, mu_v, stage_v):
    wid = lax.axis_index("s") * NC + lax.axis_index("c")
    row0 = wid * RPW

    pltpu.sync_copy(bu_h.at[pl.ds(row0, RPW)], bu_v)
    pltpu.sync_copy(bi_h.at[pl.ds(row0, RPW)], bi_v)
    pltpu.sync_copy(but_h.at[pl.ds(row0, RPW)], but_v)
    pltpu.sync_copy(bit_h.at[pl.ds(row0, RPW)], bit_v)
    pltpu.sync_copy(mu_h, mu_v)

    iota16 = lax.iota(jnp.int32, 16)
    diag = iota16 * 16
    perms = [jnp.bitwise_xor(iota16, d) for d in (8, 4, 2, 1)]

    for c in range(NCHUNK):
        off = (row0 + c * CH) * K
        pltpu.sync_copy(gu_h.at[pl.ds(off, CH * K)], gu_v)
        pltpu.sync_copy(gi_h.at[pl.ds(off, CH * K)], gi_v)
        pltpu.sync_copy(gut_h.at[pl.ds(off, CH * K)], gut_v)
        pltpu.sync_copy(git_h.at[pl.ds(off, CH * K)], git_v)

        def grp_body(g, _):
            gbase = g * 16 * K
            res = None
            for j in range(16):
                base = gbase + j * K
                acc = None
                for m in range(K // 16):
                    sl = pl.ds(base + m * 16, 16)
                    p = gu_v[sl] * gi_v[sl] + gut_v[sl] * git_v[sl]
                    acc = p if acc is None else acc + p
                for pm in perms:
                    acc = acc + acc.at[pm].get(mode="promise_in_bounds")
                res = acc if res is None else jnp.where(iota16 == j, acc, res)
            out_v[pl.ds(c * CH + g * 16, 16)] = res
            return 0

        lax.fori_loop(0, CH // 16, grp_body, 0)

    mu = mu_v[pl.ds(0, 16)]
    for q in range(RPW // 16):
        sl = pl.ds(q * 16, 16)
        out_v[sl] = (out_v[sl] + bu_v[sl] + bi_v[sl] + but_v[sl]
                     + bit_v[sl] + mu)

    pltpu.sync_copy(out_v, out_h.at[pl.ds(row0, RPW)])


def kernel(gu, gi, gut, git, bu, bi, but, bit, Mu):
    mu1 = jnp.broadcast_to(Mu.reshape(1), (16,))
    return _sc_kernel(gu.reshape(B * K), gi.reshape(B * K),
                      gut.reshape(B * K), git.reshape(B * K),
                      bu.reshape(B), bi.reshape(B),
                      but.reshape(B), bit.reshape(B), mu1)


# speedup vs baseline: 1.0139x; 1.0139x over previous
import functools

import jax
import jax.numpy as jnp
from jax import lax
from jax.experimental import pallas as pl
from jax.experimental.pallas import tpu as pltpu
from jax.experimental.pallas import tpu_sc as plsc

B = 16384
K = 64
NC = 2
NS = 16
NW = NC * NS
RPW = B // NW
CH = 128
NCHUNK = RPW // CH

_mesh = plsc.VectorSubcoreMesh(core_axis_name="c", subcore_axis_name="s")


@functools.partial(
    pl.kernel,
    mesh=_mesh,
    out_type=jax.ShapeDtypeStruct((B,), jnp.float32),
    scratch_types=[
        pltpu.VMEM((2, CH * K), jnp.float32),
        pltpu.VMEM((2, CH * K), jnp.float32),
        pltpu.VMEM((2, CH * K), jnp.float32),
        pltpu.VMEM((2, CH * K), jnp.float32),
        pltpu.VMEM((RPW,), jnp.float32),
        pltpu.VMEM((RPW,), jnp.float32),
        pltpu.VMEM((RPW,), jnp.float32),
        pltpu.VMEM((RPW,), jnp.float32),
        pltpu.VMEM((RPW,), jnp.float32),
        pltpu.VMEM((16,), jnp.float32),
        pltpu.SemaphoreType.DMA,
        pltpu.SemaphoreType.DMA,
    ],
)
def _sc_kernel(gu_h, gi_h, gut_h, git_h, bu_h, bi_h, but_h, bit_h, mu_h,
               out_h, gu_v, gi_v, gut_v, git_v,
               bu_v, bi_v, but_v, bit_v, out_v, mu_v, sem0, sem1):
    wid = lax.axis_index("s") * NC + lax.axis_index("c")
    row0 = wid * RPW
    sems = (sem0, sem1)

    pltpu.sync_copy(bu_h.at[pl.ds(row0, RPW)], bu_v)
    pltpu.sync_copy(bi_h.at[pl.ds(row0, RPW)], bi_v)
    pltpu.sync_copy(but_h.at[pl.ds(row0, RPW)], but_v)
    pltpu.sync_copy(bit_h.at[pl.ds(row0, RPW)], bit_v)
    pltpu.sync_copy(mu_h, mu_v)

    iota16 = lax.iota(jnp.int32, 16)
    perms = [jnp.bitwise_xor(iota16, d) for d in (8, 4, 2, 1)]

    def start_chunk(c):
        s = c % 2
        off = (row0 + c * CH) * K
        sem = sems[s]
        return [
            pltpu.async_copy(gu_h.at[pl.ds(off, CH * K)], gu_v.at[s], sem),
            pltpu.async_copy(gi_h.at[pl.ds(off, CH * K)], gi_v.at[s], sem),
            pltpu.async_copy(gut_h.at[pl.ds(off, CH * K)], gut_v.at[s], sem),
            pltpu.async_copy(git_h.at[pl.ds(off, CH * K)], git_v.at[s], sem),
        ]

    pending = {0: start_chunk(0)}

    for c in range(NCHUNK):
        s = c % 2
        if c + 1 < NCHUNK:
            pending[c + 1] = start_chunk(c + 1)
        for h in pending.pop(c):
            h.wait()

        def grp_body(g, _):
            res = None
            for j in range(16):
                acc = None
                base = (g * 16 + j) * K
                for m in range(K // 16):
                    sl = pl.ds(base + m * 16, 16)
                    p = (gu_v[s, sl] * gi_v[s, sl]
                         + gut_v[s, sl] * git_v[s, sl])
                    acc = p if acc is None else acc + p
                for pm in perms:
                    acc = acc + acc.at[pm].get(mode="promise_in_bounds")
                res = acc if res is None else jnp.where(iota16 == j, acc, res)
            out_v[pl.ds(c * CH + g * 16, 16)] = res
            return 0

        lax.fori_loop(0, CH // 16, grp_body, 0)

    mu = mu_v[pl.ds(0, 16)]
    for q in range(RPW // 16):
        sl = pl.ds(q * 16, 16)
        out_v[sl] = (out_v[sl] + bu_v[sl] + bi_v[sl] + but_v[sl]
                     + bit_v[sl] + mu)

    pltpu.sync_copy(out_v, out_h.at[pl.ds(row0, RPW)])


def kernel(gu, gi, gut, git, bu, bi, but, bit, Mu):
    mu1 = jnp.broadcast_to(Mu.reshape(1), (16,))
    return _sc_kernel(gu.reshape(B * K), gi.reshape(B * K),
                      gut.reshape(B * K), git.reshape(B * K),
                      bu.reshape(B), bi.reshape(B),
                      but.reshape(B), bit.reshape(B), mu1)
